# baseline (speedup 1.0000x reference)
import functools

import jax
import jax.numpy as jnp
from jax import lax
from jax.experimental import pallas as pl
from jax.experimental.pallas import tpu as pltpu
from jax.experimental.pallas import tpu_sc as plsc

B_TOTAL = 4096 * 200
D = 64
G = 128
NW = 32
ROWS_PER_W = B_TOTAL // NW
CHUNKS_PER_W = ROWS_PER_W // G

_mesh = plsc.VectorSubcoreMesh(core_axis_name="c", subcore_axis_name="s")


@functools.partial(
    pl.kernel,
    out_type=jax.ShapeDtypeStruct((B_TOTAL, D), jnp.float32),
    mesh=_mesh,
    scratch_types=[
        pltpu.VMEM((CHUNKS_PER_W, G), jnp.int32),
        pltpu.VMEM((G, D), jnp.float32),
        pltpu.SemaphoreType.DMA,
    ],
    compiler_params=pltpu.CompilerParams(use_tc_tiling_on_sc=False),
)
def _gather_kernel(idx_hbm, table_hbm, out_hbm, idx_v, rows_v, sem):
    wid = lax.axis_index("s") * 2 + lax.axis_index("c")
    pltpu.sync_copy(idx_hbm.at[pl.ds(wid * CHUNKS_PER_W, CHUNKS_PER_W)], idx_v)

    def body(j, carry):
        pltpu.async_copy(table_hbm.at[idx_v.at[j]], rows_v, sem).wait()
        row0 = (wid * CHUNKS_PER_W + j) * G
        pltpu.sync_copy(rows_v, out_hbm.at[pl.ds(row0, G)])
        return carry

    lax.fori_loop(0, CHUNKS_PER_W, body, 0)


def kernel(x, table):
    idx = x.reshape(NW * CHUNKS_PER_W, G).astype(jnp.int32)
    out = _gather_kernel(idx, table)
    return out.reshape(x.shape[0], x.shape[1], D)


# speedup vs baseline: 1.1125x; 1.1125x over previous
import functools

import jax
import jax.numpy as jnp
from jax import lax
from jax.experimental import pallas as pl
from jax.experimental.pallas import tpu as pltpu
from jax.experimental.pallas import tpu_sc as plsc

B_TOTAL = 4096 * 200
D = 64
G = 128
SUP = 512
GPS = SUP // G
NW = 32
ROWS_PER_W = B_TOTAL // NW
CHUNKS_PER_W = ROWS_PER_W // G
NSUP = ROWS_PER_W // SUP

_mesh = plsc.VectorSubcoreMesh(core_axis_name="c", subcore_axis_name="s")


@functools.partial(
    pl.kernel,
    out_type=jax.ShapeDtypeStruct((B_TOTAL, D), jnp.float32),
    mesh=_mesh,
    scratch_types=[
        pltpu.VMEM((CHUNKS_PER_W, G), jnp.int32),
        pltpu.VMEM((2, SUP, D), jnp.float32),
        pltpu.SemaphoreType.DMA,
        pltpu.SemaphoreType.DMA,
    ],
    compiler_params=pltpu.CompilerParams(use_tc_tiling_on_sc=False),
)
def _gather_kernel(idx_hbm, table_hbm, out_hbm, idx_v, rows_v, gsem, wsem):
    wid = lax.axis_index("s") * 2 + lax.axis_index("c")
    wbase = wid * ROWS_PER_W
    pltpu.sync_copy(idx_hbm.at[pl.ds(wid * CHUNKS_PER_W, CHUNKS_PER_W)], idx_v)

    def fire_gathers(sc, buf):
        return [
            pltpu.async_copy(
                table_hbm.at[idx_v.at[sc * GPS + j]],
                rows_v.at[buf, pl.ds(j * G, G)],
                gsem,
            )
            for j in range(GPS)
        ]

    def fire_wb(sc, buf):
        return pltpu.async_copy(
            rows_v.at[buf], out_hbm.at[pl.ds(wbase + sc * SUP, SUP)], wsem
        )

    def half_step(sc_gather, buf_gather):
        g = fire_gathers(sc_gather, buf_gather)
        w = fire_wb(sc_gather - 1, 1 - buf_gather)
        for cp in g:
            cp.wait()
        w.wait()

    for cp in fire_gathers(0, 0):
        cp.wait()

    def body(k, carry):
        half_step(2 * k + 1, 1)
        half_step(2 * k + 2, 0)
        return carry

    lax.fori_loop(0, (NSUP - 2) // 2, body, 0)

    half_step(NSUP - 1, 1)
    fire_wb(NSUP - 1, 1).wait()


def kernel(x, table):
    idx = x.reshape(NW * CHUNKS_PER_W, G).astype(jnp.int32)
    out = _gather_kernel(idx, table)
    return out.reshape(x.shape[0], x.shape[1], D)
